# baseline (speedup 1.0000x reference)
import functools

import jax
import jax.numpy as jnp
from jax import lax
from jax.experimental import pallas as pl
from jax.experimental.pallas import tpu as pltpu
from jax.experimental.pallas import tpu_sc as plsc

N = 10000
E = 320000
D = 128
LAYERS = 3

NC = 2
NS = 16
LANES = 16
NW = NC * NS

B = 112
NBUF = 3
IDEPTH = 2 * NBUF
NB = -(-E // (NW * B * NBUF)) * NBUF
E_PAD = NW * NB * B
NB0 = 114
NB1 = 2 * NB - NB0
NB_MAX = max(NB0, NB1)
ROWS_PER_TILE = (N // NS) // 8 * 8
ROWS_EXTRA = N - NS * ROWS_PER_TILE
VECS_PER_ROW = D // LANES

_CHUNKS = []
_off = 0
while _off < ROWS_PER_TILE:
    _sz = min(B, ROWS_PER_TILE - _off)
    _CHUNKS.append((_off, _sz))
    _off += _sz


def _spmm_body(src_hbm, col_hbm, row_hbm, val_hbm, out_hbm, acc_sh, *scr):
    rows = scr[:NBUF]
    col_b = scr[NBUF:NBUF + IDEPTH]
    row_b = scr[NBUF + IDEPTH:NBUF + 2 * IDEPTH]
    val_b = scr[NBUF + 2 * IDEPTH:NBUF + 3 * IDEPTH]
    gsem = scr[NBUF + 3 * IDEPTH:2 * NBUF + 3 * IDEPTH]
    ssem = scr[2 * NBUF + 3 * IDEPTH:3 * NBUF + 3 * IDEPTH]
    isem = scr[3 * NBUF + 3 * IDEPTH:3 * NBUF + 4 * IDEPTH]

    c = lax.axis_index("c")
    s = lax.axis_index("s")
    wid = c * NS + s
    base = s * ROWS_PER_TILE
    nb = jnp.where(c == 0, NB0, NB1)
    npair = nb // (2 * NBUF)

    def _fire_idx(j, b):
        pltpu.async_copy(col_hbm.at[wid, b], col_b[j], isem[j])
        pltpu.async_copy(row_hbm.at[wid, b], row_b[j], isem[j])
        pltpu.async_copy(val_hbm.at[wid, b], val_b[j], isem[j])

    def _wait_idx(j, b):
        pltpu.make_async_copy(col_hbm.at[wid, b], col_b[j], isem[j]).wait()
        pltpu.make_async_copy(row_hbm.at[wid, b], row_b[j], isem[j]).wait()
        pltpu.make_async_copy(val_hbm.at[wid, b], val_b[j], isem[j]).wait()

    for j in range(IDEPTH):
        _fire_idx(j, j)

    def _zero_row(r, _):
        for jj in range(VECS_PER_ROW):
            rows[0][r, pl.ds(jj * LANES, LANES)] = jnp.zeros((LANES,), jnp.float32)
        return 0
    lax.fori_loop(0, B, _zero_row, 0)
    zsem = gsem[0]
    for off, sz in _CHUNKS:
        pltpu.async_copy(rows[0].at[:sz], acc_sh.at[pl.ds(base + off, sz)], zsem)

    @pl.when(s == NS - 1)
    def _zero_tail():
        pltpu.async_copy(rows[0].at[:ROWS_EXTRA],
                         acc_sh.at[pl.ds(NS * ROWS_PER_TILE, ROWS_EXTRA)], zsem)
    for off, sz in _CHUNKS:
        pltpu.make_async_copy(
            rows[0].at[:sz], acc_sh.at[pl.ds(base + off, sz)], zsem).wait()

    @pl.when(s == NS - 1)
    def _zero_tail_wait():
        pltpu.make_async_copy(
            rows[0].at[:ROWS_EXTRA],
            acc_sh.at[pl.ds(NS * ROWS_PER_TILE, ROWS_EXTRA)], zsem).wait()

    for i in range(NBUF):
        _wait_idx(i, i)
        pltpu.async_copy(src_hbm.at[col_b[i]], rows[i], gsem[i])
    plsc.subcore_barrier()

    def _scale(buf, vref):
        def _scale_group(g, _):
            vvec = vref[pl.ds(g * LANES, LANES)]
            for k in range(LANES):
                r = g * LANES + k
                v = vvec[k]
                for jj in range(VECS_PER_ROW):
                    sl = pl.ds(jj * LANES, LANES)
                    buf[r, sl] = buf[r, sl] * v
            return 0
        lax.fori_loop(0, B // LANES, _scale_group, 0)

    def _pair(h, _):
        b_pair = h * 2 * NBUF
        for sub in range(2):
            for i in range(NBUF):
                j = sub * NBUF + i
                b = b_pair + j
                pltpu.make_async_copy(src_hbm.at[col_b[j]], rows[i], gsem[i]).wait()
                _scale(rows[i], val_b[j])
                if i > 0:
                    jp = sub * NBUF + i - 1
                    pltpu.make_async_copy(
                        rows[i - 1], acc_sh.at[row_b[jp]], ssem[i - 1]).wait()
                pltpu.async_copy(rows[i], acc_sh.at[row_b[j]], ssem[i], add=True)
            for i in range(NBUF):
                j = sub * NBUF + i
                jn = (j + NBUF) % IDEPTH
                b = b_pair + j
                if i == NBUF - 1:
                    pltpu.make_async_copy(
                        rows[i], acc_sh.at[row_b[j]], ssem[i]).wait()

                @pl.when(b + NBUF < nb)
                def _next_gather():
                    _wait_idx(jn, b + NBUF)
                    pltpu.async_copy(src_hbm.at[col_b[jn]], rows[i], gsem[i])
            for i in range(NBUF):
                j = sub * NBUF + i
                b = b_pair + j

                @pl.when(b + IDEPTH < nb)
                def _next_idx():
                    _fire_idx(j, b + IDEPTH)
        return 0
    lax.fori_loop(0, npair, _pair, 0)

    plsc.subcore_barrier()

    wsem = gsem[1]
    for off, sz in _CHUNKS:
        pltpu.async_copy(acc_sh.at[pl.ds(base + off, sz)],
                         out_hbm.at[c, pl.ds(base + off, sz)], wsem)

    @pl.when(s == NS - 1)
    def _write_tail():
        tail = NS * ROWS_PER_TILE
        pltpu.async_copy(acc_sh.at[pl.ds(tail, ROWS_EXTRA)],
                         out_hbm.at[c, pl.ds(tail, ROWS_EXTRA)], wsem)
    for off, sz in _CHUNKS:
        pltpu.make_async_copy(acc_sh.at[pl.ds(base + off, sz)],
                              out_hbm.at[c, pl.ds(base + off, sz)], wsem).wait()

    @pl.when(s == NS - 1)
    def _write_tail_wait():
        tail = NS * ROWS_PER_TILE
        pltpu.make_async_copy(acc_sh.at[pl.ds(tail, ROWS_EXTRA)],
                              out_hbm.at[c, pl.ds(tail, ROWS_EXTRA)], wsem).wait()


_spmm = functools.partial(
    pl.kernel,
    mesh=plsc.VectorSubcoreMesh(core_axis_name="c", subcore_axis_name="s"),
    out_type=jax.ShapeDtypeStruct((NC, N, D), jnp.float32),
    scratch_types=[pltpu.VMEM_SHARED((N, D), jnp.float32)]
    + [pltpu.VMEM((B, D), jnp.float32) for _ in range(NBUF)]
    + [pltpu.VMEM((B,), jnp.int32) for _ in range(IDEPTH)]
    + [pltpu.VMEM((B,), jnp.int32) for _ in range(IDEPTH)]
    + [pltpu.VMEM((B,), jnp.float32) for _ in range(IDEPTH)]
    + [pltpu.SemaphoreType.DMA for _ in range(2 * NBUF + IDEPTH)],
)(_spmm_body)


def _merge_body(parts_ref, base_ref, y_ref, acc_ref, *, out_scale):
    y = parts_ref[0] + parts_ref[1]
    nrm = jnp.maximum(jnp.sqrt(jnp.sum(y * y, axis=1, keepdims=True)), 1e-12)
    y_ref[...] = y
    acc_ref[...] = (base_ref[...] + y / nrm) * out_scale


_MBLK = 2000


def _merge(parts, base, out_scale):
    return pl.pallas_call(
        functools.partial(_merge_body, out_scale=out_scale),
        grid=(N // _MBLK,),
        in_specs=[
            pl.BlockSpec((NC, _MBLK, D), lambda i: (0, i, 0)),
            pl.BlockSpec((_MBLK, D), lambda i: (i, 0)),
        ],
        out_specs=[
            pl.BlockSpec((_MBLK, D), lambda i: (i, 0)),
            pl.BlockSpec((_MBLK, D), lambda i: (i, 0)),
        ],
        out_shape=[
            jax.ShapeDtypeStruct((N, D), jnp.float32),
            jax.ShapeDtypeStruct((N, D), jnp.float32),
        ],
    )(parts, base)


def _arrange(x):
    x = jnp.pad(x, (0, E_PAD - E))
    sc0 = x[:NS * NB0 * B].reshape(NS, NB0, B)
    sc0 = jnp.pad(sc0, ((0, 0), (0, NB_MAX - NB0), (0, 0)))
    sc1 = x[NS * NB0 * B:].reshape(NS, NB1, B)
    sc1 = jnp.pad(sc1, ((0, 0), (0, NB_MAX - NB1), (0, 0)))
    return jnp.concatenate([sc0, sc1], axis=0)


def kernel(ego_embeddings, nei_embeddings, edge_index, A_values):
    col = _arrange(edge_index[1])
    row = _arrange(edge_index[0])
    val = _arrange(A_values)

    src = nei_embeddings
    acc = ego_embeddings
    for layer in range(LAYERS):
        parts = _spmm(src, col, row, val)
        scale = 1.0 / (LAYERS + 1) if layer == LAYERS - 1 else 1.0
        src, acc = _merge(parts, acc, scale)
    return acc


# speedup vs baseline: 1.0770x; 1.0770x over previous
import functools

import jax
import jax.numpy as jnp
from jax import lax
from jax.experimental import pallas as pl
from jax.experimental.pallas import tpu as pltpu
from jax.experimental.pallas import tpu_sc as plsc

N = 10000
E = 320000
D = 128
LAYERS = 3

NC = 2
NS = 16
LANES = 16
NW = NC * NS

B = 112
NBUF = 3
IDEPTH = 2 * NBUF
NB = -(-E // (NW * B * NBUF)) * NBUF
E_PAD = NW * NB * B
NB0 = 126
NB1 = 2 * NB - NB0
NB_MAX = max(NB0, NB1)
ROWS_PER_TILE = (N // NS) // 8 * 8
ROWS_EXTRA = N - NS * ROWS_PER_TILE
VECS_PER_ROW = D // LANES

_CHUNKS = []
_off = 0
while _off < ROWS_PER_TILE:
    _sz = min(B, ROWS_PER_TILE - _off)
    _CHUNKS.append((_off, _sz))
    _off += _sz


def _spmm_body(src_hbm, col_hbm, row_hbm, val_hbm, out_hbm, acc_sh, *scr):
    rows = scr[:NBUF]
    col_b = scr[NBUF:NBUF + IDEPTH]
    row_b = scr[NBUF + IDEPTH:NBUF + 2 * IDEPTH]
    val_b = scr[NBUF + 2 * IDEPTH:NBUF + 3 * IDEPTH]
    gsem = scr[NBUF + 3 * IDEPTH:2 * NBUF + 3 * IDEPTH]
    ssem = scr[2 * NBUF + 3 * IDEPTH:3 * NBUF + 3 * IDEPTH]
    isem = scr[3 * NBUF + 3 * IDEPTH:3 * NBUF + 4 * IDEPTH]

    c = lax.axis_index("c")
    s = lax.axis_index("s")
    wid = c * NS + s
    base = s * ROWS_PER_TILE
    nb = jnp.where(c == 0, NB0, NB1)
    npair = nb // (2 * NBUF)

    def _fire_idx(j, b):
        pltpu.async_copy(col_hbm.at[wid, b], col_b[j], isem[j])
        pltpu.async_copy(row_hbm.at[wid, b], row_b[j], isem[j])
        pltpu.async_copy(val_hbm.at[wid, b], val_b[j], isem[j])

    def _wait_idx(j, b):
        pltpu.make_async_copy(col_hbm.at[wid, b], col_b[j], isem[j]).wait()
        pltpu.make_async_copy(row_hbm.at[wid, b], row_b[j], isem[j]).wait()
        pltpu.make_async_copy(val_hbm.at[wid, b], val_b[j], isem[j]).wait()

    for j in range(IDEPTH):
        _fire_idx(j, j)

    def _zero_row(r, _):
        for jj in range(VECS_PER_ROW):
            rows[0][r, pl.ds(jj * LANES, LANES)] = jnp.zeros((LANES,), jnp.float32)
        return 0
    lax.fori_loop(0, B, _zero_row, 0)
    zsem = gsem[0]
    for off, sz in _CHUNKS:
        pltpu.async_copy(rows[0].at[:sz], acc_sh.at[pl.ds(base + off, sz)], zsem)

    @pl.when(s == NS - 1)
    def _zero_tail():
        pltpu.async_copy(rows[0].at[:ROWS_EXTRA],
                         acc_sh.at[pl.ds(NS * ROWS_PER_TILE, ROWS_EXTRA)], zsem)
    for off, sz in _CHUNKS:
        pltpu.make_async_copy(
            rows[0].at[:sz], acc_sh.at[pl.ds(base + off, sz)], zsem).wait()

    @pl.when(s == NS - 1)
    def _zero_tail_wait():
        pltpu.make_async_copy(
            rows[0].at[:ROWS_EXTRA],
            acc_sh.at[pl.ds(NS * ROWS_PER_TILE, ROWS_EXTRA)], zsem).wait()

    for i in range(NBUF):
        _wait_idx(i, i)
        pltpu.async_copy(src_hbm.at[col_b[i]], rows[i], gsem[i])
    plsc.subcore_barrier()

    def _scale(buf, vref):
        def _scale_group(g, _):
            vvec = vref[pl.ds(g * LANES, LANES)]
            for k in range(LANES):
                r = g * LANES + k
                v = vvec[k]
                for jj in range(VECS_PER_ROW):
                    sl = pl.ds(jj * LANES, LANES)
                    buf[r, sl] = buf[r, sl] * v
            return 0
        lax.fori_loop(0, B // LANES, _scale_group, 0)

    def _pair(h, _):
        b_pair = h * 2 * NBUF
        for sub in range(2):
            for i in range(NBUF):
                j = sub * NBUF + i
                b = b_pair + j
                pltpu.make_async_copy(src_hbm.at[col_b[j]], rows[i], gsem[i]).wait()
                _scale(rows[i], val_b[j])
                if i > 0:
                    jp = sub * NBUF + i - 1
                    pltpu.make_async_copy(
                        rows[i - 1], acc_sh.at[row_b[jp]], ssem[i - 1]).wait()
                pltpu.async_copy(rows[i], acc_sh.at[row_b[j]], ssem[i], add=True)
            for i in range(NBUF):
                j = sub * NBUF + i
                jn = (j + NBUF) % IDEPTH
                b = b_pair + j
                if i == NBUF - 1:
                    pltpu.make_async_copy(
                        rows[i], acc_sh.at[row_b[j]], ssem[i]).wait()

                @pl.when(b + NBUF < nb)
                def _next_gather():
                    _wait_idx(jn, b + NBUF)
                    pltpu.async_copy(src_hbm.at[col_b[jn]], rows[i], gsem[i])
            for i in range(NBUF):
                j = sub * NBUF + i
                b = b_pair + j

                @pl.when(b + IDEPTH < nb)
                def _next_idx():
                    _fire_idx(j, b + IDEPTH)
        return 0
    lax.fori_loop(0, npair, _pair, 0)

    plsc.subcore_barrier()

    wsem = gsem[1]
    for off, sz in _CHUNKS:
        pltpu.async_copy(acc_sh.at[pl.ds(base + off, sz)],
                         out_hbm.at[c, pl.ds(base + off, sz)], wsem)

    @pl.when(s == NS - 1)
    def _write_tail():
        tail = NS * ROWS_PER_TILE
        pltpu.async_copy(acc_sh.at[pl.ds(tail, ROWS_EXTRA)],
                         out_hbm.at[c, pl.ds(tail, ROWS_EXTRA)], wsem)
    for off, sz in _CHUNKS:
        pltpu.make_async_copy(acc_sh.at[pl.ds(base + off, sz)],
                              out_hbm.at[c, pl.ds(base + off, sz)], wsem).wait()

    @pl.when(s == NS - 1)
    def _write_tail_wait():
        tail = NS * ROWS_PER_TILE
        pltpu.make_async_copy(acc_sh.at[pl.ds(tail, ROWS_EXTRA)],
                              out_hbm.at[c, pl.ds(tail, ROWS_EXTRA)], wsem).wait()


_spmm = functools.partial(
    pl.kernel,
    mesh=plsc.VectorSubcoreMesh(core_axis_name="c", subcore_axis_name="s"),
    out_type=jax.ShapeDtypeStruct((NC, N, D), jnp.float32),
    scratch_types=[pltpu.VMEM_SHARED((N, D), jnp.float32)]
    + [pltpu.VMEM((B, D), jnp.float32) for _ in range(NBUF)]
    + [pltpu.VMEM((B,), jnp.int32) for _ in range(IDEPTH)]
    + [pltpu.VMEM((B,), jnp.int32) for _ in range(IDEPTH)]
    + [pltpu.VMEM((B,), jnp.float32) for _ in range(IDEPTH)]
    + [pltpu.SemaphoreType.DMA for _ in range(2 * NBUF + IDEPTH)],
)(_spmm_body)


def _merge_body(parts_ref, base_ref, y_ref, acc_ref, *, out_scale):
    y = parts_ref[0] + parts_ref[1]
    nrm = jnp.maximum(jnp.sqrt(jnp.sum(y * y, axis=1, keepdims=True)), 1e-12)
    y_ref[...] = y
    acc_ref[...] = (base_ref[...] + y / nrm) * out_scale


_MBLK = 2000


def _merge(parts, base, out_scale):
    return pl.pallas_call(
        functools.partial(_merge_body, out_scale=out_scale),
        grid=(N // _MBLK,),
        in_specs=[
            pl.BlockSpec((NC, _MBLK, D), lambda i: (0, i, 0)),
            pl.BlockSpec((_MBLK, D), lambda i: (i, 0)),
        ],
        out_specs=[
            pl.BlockSpec((_MBLK, D), lambda i: (i, 0)),
            pl.BlockSpec((_MBLK, D), lambda i: (i, 0)),
        ],
        out_shape=[
            jax.ShapeDtypeStruct((N, D), jnp.float32),
            jax.ShapeDtypeStruct((N, D), jnp.float32),
        ],
    )(parts, base)


def _arrange(x):
    x = jnp.pad(x, (0, E_PAD - E))
    sc0 = x[:NS * NB0 * B].reshape(NS, NB0, B)
    sc0 = jnp.pad(sc0, ((0, 0), (0, NB_MAX - NB0), (0, 0)))
    sc1 = x[NS * NB0 * B:].reshape(NS, NB1, B)
    sc1 = jnp.pad(sc1, ((0, 0), (0, NB_MAX - NB1), (0, 0)))
    return jnp.concatenate([sc0, sc1], axis=0)


def kernel(ego_embeddings, nei_embeddings, edge_index, A_values):
    col = _arrange(edge_index[1])
    row = _arrange(edge_index[0])
    val = _arrange(A_values)

    src = nei_embeddings
    acc = ego_embeddings
    for layer in range(LAYERS):
        parts = _spmm(src, col, row, val)
        scale = 1.0 / (LAYERS + 1) if layer == LAYERS - 1 else 1.0
        src, acc = _merge(parts, acc, scale)
    return acc
